# Initial kernel scaffold; baseline (speedup 1.0000x reference)
import jax
import jax.numpy as jnp
from jax.experimental import pallas as pl


def kernel(x, weights):
    raise NotImplementedError("write your pallas kernel here")


# speedup vs baseline: 29.0836x; 29.0836x over previous
import jax
import jax.numpy as jnp
from jax.experimental import pallas as pl


def _som_dist_kernel(x_ref, w_ref, out_ref):
    x = x_ref[...]
    w = w_ref[...]
    xw = jax.lax.dot_general(
        x, w, (((1,), (1,)), ((), ())),
        preferred_element_type=jnp.float32,
    )
    x2 = jnp.sum(x * x, axis=1, keepdims=True)
    w2 = jnp.sum(w * w, axis=1, keepdims=True).T
    out_ref[...] = (x2 - 2.0 * xw) + w2


def kernel(x, weights):
    R, C, D = weights.shape
    B = x.shape[0]
    w2d = weights.reshape(R * C, D)
    out = pl.pallas_call(
        _som_dist_kernel,
        out_shape=jax.ShapeDtypeStruct((B, R * C), jnp.float32),
    )(x, w2d)
    return out.reshape(B, R, C)
